# baseline (speedup 1.0000x reference)
import jax
import jax.numpy as jnp
from jax import lax
from jax.experimental import pallas as pl
from jax.experimental.pallas import tpu as pltpu
from jax.experimental.pallas import tpu_sc as plsc

RES = 128
FEAT = 8
LANES = 16

NC = 2
NS = 16
NW = NC * NS

NPTS = 2_000_000
CHUNK = 512
NCHUNK_TOT = (NPTS + CHUNK - 1) // CHUNK
NCW = (NCHUNK_TOT + NW - 1) // NW
SUB = 128
NSUB = CHUNK // SUB

CORNER_OFF = tuple((dx * RES + dy) * RES + dz
                   for dx in (0, 1) for dy in (0, 1) for dz in (0, 1))


STAGE_K = 256


def _tec_body(pts_hbm, feat_hbm, out_hbm, table_hbm,
              pts_v, fx_v, fy_v, fz_v, idx_v, corner_v, out_v,
              stage_in, stage_out, gsem):
    cid = lax.axis_index("c")
    sid = lax.axis_index("s")
    wid = sid * NC + cid

    stage_rows = (RES ** 3 * FEAT // 128) // NS
    rpf = 128 // FEAT

    iota_st = lax.iota(jnp.int32, LANES)
    bsel_st = iota_st >> 3
    flane_st = iota_st & 7

    def stage(i, _):
        r = sid * stage_rows + i * STAGE_K
        pltpu.sync_copy(feat_hbm.at[pl.ds(r, STAGE_K), :], stage_in)

        def mov(m, _):
            v = stage_in[m >> 3, pl.ds((m & 7) * LANES, LANES)]
            plsc.store_scatter(stage_out, [2 * m + bsel_st, flane_st], v)
            return 0

        lax.fori_loop(0, STAGE_K * 8, mov, 0, unroll=4)
        pltpu.sync_copy(
            stage_out, table_hbm.at[cid, pl.ds(r * rpf, STAGE_K * rpf), :])
        return 0

    lax.fori_loop(0, stage_rows // STAGE_K, stage, 0)
    plsc.subcore_barrier()

    iota = lax.iota(jnp.int32, LANES)
    bsel = iota >> 3
    feat_lane = iota & 7
    zeros16 = jnp.zeros((LANES,), jnp.int32)
    ones16 = zeros16 + 1
    twos16 = zeros16 + 2

    def chunk_body(t, _):
        g = t * NW + wid

        @pl.when(g < NCHUNK_TOT)
        def _():
            base = jnp.minimum(g * CHUNK, NPTS - CHUNK)
            pltpu.sync_copy(
                pts_hbm.at[pl.ds(base * 3 // 128, CHUNK * 3 // 128), :], pts_v)

            def grp(gg, _):
                rows3 = (gg * LANES + iota) * 3
                px = plsc.load_gather(pts_v, [rows3 >> 7, rows3 & 127])
                py = plsc.load_gather(pts_v, [(rows3 + 1) >> 7, (rows3 + 1) & 127])
                pz = plsc.load_gather(pts_v, [(rows3 + 2) >> 7, (rows3 + 2) & 127])

                def split(p):
                    x = (p + 1.0) * (0.5 * (RES - 1))
                    xi = jnp.clip(x.astype(jnp.int32), 0, RES - 2)
                    return xi, x - xi.astype(jnp.float32)

                xi, fx = split(px)
                yi, fy = split(py)
                zi, fz = split(pz)
                off = gg * LANES
                fx_v[pl.ds(off, LANES)] = fx
                fy_v[pl.ds(off, LANES)] = fy
                fz_v[pl.ds(off, LANES)] = fz
                flat = (xi * RES + yi) * RES + zi
                j = gg >> 3
                o = (gg & 7) * LANES
                for c in range(8):
                    idx_v[c, j, pl.ds(o, LANES)] = flat + CORNER_OFF[c]
                return 0

            lax.fori_loop(0, CHUNK // LANES, grp, 0, unroll=2)

            copies = []
            for c in range(8):
                for j in range(NSUB):
                    copies.append(pltpu.async_copy(
                        table_hbm.at[cid].at[idx_v.at[c, j]],
                        corner_v.at[c, pl.ds(j * SUB, SUB), :],
                        gsem))
            for cp in copies:
                cp.wait()

            def pair(p, _):
                rows2 = 2 * p + bsel
                fxb = plsc.load_gather(fx_v, [rows2])
                fyb = plsc.load_gather(fy_v, [rows2])
                fzb = plsc.load_gather(fz_v, [rows2])
                wx = (1.0 - fxb, fxb)
                wy = (1.0 - fyb, fyb)
                wz = (1.0 - fzb, fzb)
                acc = jnp.zeros((LANES,), jnp.float32)
                c = 0
                for dx in (0, 1):
                    for dy in (0, 1):
                        wxy = wx[dx] * wy[dy]
                        for dz in (0, 1):
                            w = wxy * wz[dz]
                            corner = plsc.load_gather(
                                corner_v, [zeros16 + c, rows2, feat_lane])
                            acc = acc + corner * w
                            c += 1
                out_v[p >> 3, pl.ds((p & 7) * LANES, LANES)] = acc
                return 0

            lax.fori_loop(0, CHUNK // 2, pair, 0, unroll=2)

            pltpu.sync_copy(
                out_v,
                out_hbm.at[pl.ds(base * FEAT // 128, CHUNK * FEAT // 128), :])

        return 0

    lax.fori_loop(0, NCW, chunk_body, 0)


@jax.jit
def _lod_interp(pts_flat, features):
    mesh = plsc.VectorSubcoreMesh(core_axis_name="c", subcore_axis_name="s")
    run = pl.kernel(
        _tec_body,
        out_type=(
            jax.ShapeDtypeStruct((NPTS * FEAT // 128, 128), jnp.float32),
            jax.ShapeDtypeStruct((NC, RES ** 3, FEAT), jnp.float32),
        ),
        mesh=mesh,
        compiler_params=pltpu.CompilerParams(
            needs_layout_passes=False,
            use_tc_tiling_on_sc=False,
        ),
        scratch_types=[
            pltpu.VMEM((CHUNK * 3 // 128, 128), jnp.float32),
            pltpu.VMEM((CHUNK,), jnp.float32),
            pltpu.VMEM((CHUNK,), jnp.float32),
            pltpu.VMEM((CHUNK,), jnp.float32),
            pltpu.VMEM((8, NSUB, SUB), jnp.int32),
            pltpu.VMEM((8, CHUNK, FEAT), jnp.float32),
            pltpu.VMEM((CHUNK * FEAT // 128, 128), jnp.float32),
            pltpu.VMEM((STAGE_K, 128), jnp.float32),
            pltpu.VMEM((STAGE_K * 128 // FEAT, FEAT), jnp.float32),
            pltpu.SemaphoreType.DMA,
        ],
    )
    return run(pts_flat, features)


def kernel(pts, features):
    n = pts.shape[0]
    v = features.shape[0]
    out, _ = _lod_interp(pts.reshape(n * 3 // 128, 128),
                         features.reshape(v * FEAT // 128, 128))
    return out.reshape(n, FEAT)


# speedup vs baseline: 4.9700x; 4.9700x over previous
import jax
import jax.numpy as jnp
from jax import lax
from jax.experimental import pallas as pl
from jax.experimental.pallas import tpu as pltpu
from jax.experimental.pallas import tpu_sc as plsc

RES = 128
FEAT = 8
LANES = 16

NC = 2
NS = 16
NW = NC * NS

NPTS = 2_000_000
CHUNK = 512
NCHUNK_TOT = (NPTS + CHUNK - 1) // CHUNK
NCW = (NCHUNK_TOT + NW - 1) // NW
SUB = 128
NSUB = CHUNK // SUB

STAGE_K = 256

CORNER_OFF = tuple((dx * RES + dy) * RES + dz
                   for dx in (0, 1) for dy in (0, 1) for dz in (0, 1))


def _tec_body(px_hbm, py_hbm, pz_hbm, feat_hbm, out_hbm, table_hbm,
              px_v, py_v, pz_v, fx_v, fy_v, fz_v, idx_v, corner_v, out_v,
              stage_in, stage_out, gsem):
    cid = lax.axis_index("c")
    sid = lax.axis_index("s")
    wid = sid * NC + cid

    iota = lax.iota(jnp.int32, LANES)
    bsel = iota >> 3
    feat_lane = iota & 7
    zeros16 = jnp.zeros((LANES,), jnp.int32)

    stage_rows = (RES ** 3 * FEAT // 128) // NS
    vox_per_blk = STAGE_K // FEAT * 128

    def stage(i, _):
        r0 = sid * stage_rows + i * STAGE_K
        pltpu.sync_copy(feat_hbm.at[pl.ds(r0, STAGE_K), :], stage_in)

        def mov(m, _):
            r = m >> 3
            z0 = (m & 7) * LANES
            v = stage_in[r, pl.ds(z0, LANES)]
            rows = (r >> 3) * 128 + z0 + iota
            plsc.store_scatter(stage_out, [rows, (r & 7) + zeros16], v)
            return 0

        lax.fori_loop(0, STAGE_K * 8, mov, 0, unroll=4)
        pltpu.sync_copy(
            stage_out,
            table_hbm.at[cid, pl.ds(r0 // FEAT * 128, vox_per_blk), :])
        return 0

    lax.fori_loop(0, stage_rows // STAGE_K, stage, 0)
    plsc.subcore_barrier()

    def chunk_body(t, _):
        g = t * NW + wid

        @pl.when(g < NCHUNK_TOT)
        def _():
            base = jnp.minimum(g * CHUNK, NPTS - CHUNK)
            pltpu.sync_copy(px_hbm.at[pl.ds(base, CHUNK)], px_v)
            pltpu.sync_copy(py_hbm.at[pl.ds(base, CHUNK)], py_v)
            pltpu.sync_copy(pz_hbm.at[pl.ds(base, CHUNK)], pz_v)

            def grp(gg, _):
                off = gg * LANES
                px = px_v[pl.ds(off, LANES)]
                py = py_v[pl.ds(off, LANES)]
                pz = pz_v[pl.ds(off, LANES)]

                def split(p):
                    x = (p + 1.0) * (0.5 * (RES - 1))
                    xi = jnp.clip(x.astype(jnp.int32), 0, RES - 2)
                    return xi, x - xi.astype(jnp.float32)

                xi, fx = split(px)
                yi, fy = split(py)
                zi, fz = split(pz)
                fx_v[pl.ds(off, LANES)] = fx
                fy_v[pl.ds(off, LANES)] = fy
                fz_v[pl.ds(off, LANES)] = fz
                flat = (xi * RES + yi) * RES + zi
                j = gg >> 3
                o = (gg & 7) * LANES
                for c in range(8):
                    idx_v[c, j, pl.ds(o, LANES)] = flat + CORNER_OFF[c]
                return 0

            lax.fori_loop(0, CHUNK // LANES, grp, 0, unroll=2)

            copies = []
            for c in range(8):
                for j in range(NSUB):
                    copies.append(pltpu.async_copy(
                        table_hbm.at[cid].at[idx_v.at[c, j]],
                        corner_v.at[c, pl.ds(j * SUB, SUB), :],
                        gsem))
            for cp in copies:
                cp.wait()

            def pair(p, _):
                rows2 = 2 * p + bsel
                fxb = plsc.load_gather(fx_v, [rows2])
                fyb = plsc.load_gather(fy_v, [rows2])
                fzb = plsc.load_gather(fz_v, [rows2])
                wx = (1.0 - fxb, fxb)
                wy = (1.0 - fyb, fyb)
                wz = (1.0 - fzb, fzb)
                acc = jnp.zeros((LANES,), jnp.float32)
                c = 0
                for dx in (0, 1):
                    for dy in (0, 1):
                        wxy = wx[dx] * wy[dy]
                        for dz in (0, 1):
                            w = wxy * wz[dz]
                            corner = plsc.load_gather(
                                corner_v, [zeros16 + c, rows2, feat_lane])
                            acc = acc + corner * w
                            c += 1
                rows = ((p >> 6) << 3) + feat_lane
                cols = 2 * (p & 63) + bsel
                plsc.store_scatter(out_v, [rows, cols], acc)
                return 0

            lax.fori_loop(0, CHUNK // 2, pair, 0, unroll=2)

            pltpu.sync_copy(
                out_v,
                out_hbm.at[pl.ds(base // 128 * FEAT, CHUNK // 128 * FEAT), :])

        return 0

    lax.fori_loop(0, NCW, chunk_body, 0)


@jax.jit
def _lod_interp(px, py, pz, feat_lin):
    mesh = plsc.VectorSubcoreMesh(core_axis_name="c", subcore_axis_name="s")
    run = pl.kernel(
        _tec_body,
        out_type=(
            jax.ShapeDtypeStruct((NPTS * FEAT // 128, 128), jnp.float32),
            jax.ShapeDtypeStruct((NC, RES ** 3, FEAT), jnp.float32),
        ),
        mesh=mesh,
        compiler_params=pltpu.CompilerParams(
            needs_layout_passes=False,
            use_tc_tiling_on_sc=False,
        ),
        scratch_types=[
            pltpu.VMEM((CHUNK,), jnp.float32),
            pltpu.VMEM((CHUNK,), jnp.float32),
            pltpu.VMEM((CHUNK,), jnp.float32),
            pltpu.VMEM((CHUNK,), jnp.float32),
            pltpu.VMEM((CHUNK,), jnp.float32),
            pltpu.VMEM((CHUNK,), jnp.float32),
            pltpu.VMEM((8, NSUB, SUB), jnp.int32),
            pltpu.VMEM((8, CHUNK, FEAT), jnp.float32),
            pltpu.VMEM((CHUNK * FEAT // 128, 128), jnp.float32),
            pltpu.VMEM((STAGE_K, 128), jnp.float32),
            pltpu.VMEM((STAGE_K * 128 // FEAT, FEAT), jnp.float32),
            pltpu.SemaphoreType.DMA,
        ],
    )
    return run(px, py, pz, feat_lin)


def kernel(pts, features):
    n = pts.shape[0]
    v = features.shape[0]
    feat_lin = (features.reshape(v // 128, 128, FEAT)
                .transpose(0, 2, 1)
                .reshape(v * FEAT // 128, 128))
    out128, _ = _lod_interp(pts[:, 0], pts[:, 1], pts[:, 2], feat_lin)
    return (out128.reshape(n // 128, FEAT, 128)
            .transpose(0, 2, 1)
            .reshape(n, FEAT))


# speedup vs baseline: 7.0731x; 1.4231x over previous
import jax
import jax.numpy as jnp
from jax import lax
from jax.experimental import pallas as pl
from jax.experimental.pallas import tpu as pltpu
from jax.experimental.pallas import tpu_sc as plsc

RES = 128
FEAT = 8
LANES = 16

NC = 2
NS = 16
NW = NC * NS

NPTS = 2_000_000
CHUNK = 512
NCHUNK_TOT = (NPTS + CHUNK - 1) // CHUNK
NCW = (NCHUNK_TOT + NW - 1) // NW
NCW2 = (NCW + 1) // 2
SUB = 128
NSUB = CHUNK // SUB

STAGE_K = 64

CORNER_OFF = tuple((dx * RES + dy) * RES + dz
                   for dx in (0, 1) for dy in (0, 1) for dz in (0, 1))


def _tec_body(px_hbm, py_hbm, pz_hbm, feat_hbm, out_hbm, table_hbm,
              px_v, py_v, pz_v, fx_v, fy_v, fz_v, idx_v, corner_v, out_v,
              stage_in, stage_out, sem0, sem1):
    cid = lax.axis_index("c")
    sid = lax.axis_index("s")
    wid = sid * NC + cid
    sems = (sem0, sem1)

    iota = lax.iota(jnp.int32, LANES)
    bsel = iota >> 3
    feat_lane = iota & 7
    zeros16 = jnp.zeros((LANES,), jnp.int32)

    stage_rows = (RES ** 3 * FEAT // 128) // NS
    vox_per_blk = STAGE_K // FEAT * 128

    def stage(i, _):
        r0 = sid * stage_rows + i * STAGE_K
        pltpu.sync_copy(feat_hbm.at[pl.ds(r0, STAGE_K), :], stage_in)

        def mov(m, _):
            r = m >> 3
            z0 = (m & 7) * LANES
            v = stage_in[r, pl.ds(z0, LANES)]
            rows = (r >> 3) * 128 + z0 + iota
            plsc.store_scatter(stage_out, [rows, (r & 7) + zeros16], v)
            return 0

        lax.fori_loop(0, STAGE_K * 8, mov, 0, unroll=4)
        pltpu.sync_copy(
            stage_out,
            table_hbm.at[cid, pl.ds(r0 // FEAT * 128, vox_per_blk), :])
        return 0

    lax.fori_loop(0, stage_rows // STAGE_K, stage, 0)
    plsc.subcore_barrier()

    def prefetch(slot, g):
        base = jnp.minimum(g * CHUNK, NPTS - CHUNK)
        pltpu.sync_copy(px_hbm.at[pl.ds(base, CHUNK)], px_v)
        pltpu.sync_copy(py_hbm.at[pl.ds(base, CHUNK)], py_v)
        pltpu.sync_copy(pz_hbm.at[pl.ds(base, CHUNK)], pz_v)

        def grp(gg, _):
            off = gg * LANES
            px = px_v[pl.ds(off, LANES)]
            py = py_v[pl.ds(off, LANES)]
            pz = pz_v[pl.ds(off, LANES)]

            def split(p):
                x = (p + 1.0) * (0.5 * (RES - 1))
                xi = jnp.clip(x.astype(jnp.int32), 0, RES - 2)
                return xi, x - xi.astype(jnp.float32)

            xi, fx = split(px)
            yi, fy = split(py)
            zi, fz = split(pz)
            fx_v[slot, pl.ds(off, LANES)] = fx
            fy_v[slot, pl.ds(off, LANES)] = fy
            fz_v[slot, pl.ds(off, LANES)] = fz
            flat = (xi * RES + yi) * RES + zi
            j = gg >> 3
            o = (gg & 7) * LANES
            for c in range(8):
                idx_v[slot, c, j, pl.ds(o, LANES)] = flat + CORNER_OFF[c]
            return 0

        lax.fori_loop(0, CHUNK // LANES, grp, 0, unroll=2)

        for c in range(8):
            for j in range(NSUB):
                pltpu.async_copy(
                    table_hbm.at[cid].at[idx_v.at[slot, c, j]],
                    corner_v.at[slot, c, pl.ds(j * SUB, SUB), :],
                    sems[slot])

    def drain(slot):
        for c in range(8):
            for j in range(NSUB):
                pltpu.make_async_copy(
                    table_hbm.at[cid].at[idx_v.at[slot, c, j]],
                    corner_v.at[slot, c, pl.ds(j * SUB, SUB), :],
                    sems[slot]).wait()

    def blend(slot, g):
        base = jnp.minimum(g * CHUNK, NPTS - CHUNK)

        def pair(p, _):
            rows2 = 2 * p + bsel
            fxb = plsc.load_gather(fx_v.at[slot], [rows2])
            fyb = plsc.load_gather(fy_v.at[slot], [rows2])
            fzb = plsc.load_gather(fz_v.at[slot], [rows2])
            wx = (1.0 - fxb, fxb)
            wy = (1.0 - fyb, fyb)
            acc = jnp.zeros((LANES,), jnp.float32)
            for dx in (0, 1):
                for dy in (0, 1):
                    wxy = wx[dx] * wy[dy]
                    c = (dx * 2 + dy) * 2
                    f0 = plsc.load_gather(
                        corner_v.at[slot], [zeros16 + c, rows2, feat_lane])
                    f1 = plsc.load_gather(
                        corner_v.at[slot], [zeros16 + c + 1, rows2, feat_lane])
                    cz = f0 + fzb * (f1 - f0)
                    acc = acc + wxy * cz
            rows = ((p >> 6) << 3) + feat_lane
            cols = 2 * (p & 63) + bsel
            plsc.store_scatter(out_v, [rows, cols], acc)
            return 0

        lax.fori_loop(0, CHUNK // 2, pair, 0, unroll=2)

        pltpu.sync_copy(
            out_v,
            out_hbm.at[pl.ds(base // 128 * FEAT, CHUNK // 128 * FEAT), :])

    g0 = wid

    @pl.when(g0 < NCHUNK_TOT)
    def _():
        prefetch(0, g0)

    def chunk_pipe(tt, _):
        t0 = 2 * tt
        for phase in (0, 1):
            g_cur = (t0 + phase) * NW + wid
            g_nxt = (t0 + phase + 1) * NW + wid

            @pl.when(g_nxt < NCHUNK_TOT)
            def _():
                prefetch(1 - phase, g_nxt)

            @pl.when(g_cur < NCHUNK_TOT)
            def _():
                drain(phase)
                blend(phase, g_cur)
        return 0

    lax.fori_loop(0, NCW2, chunk_pipe, 0)


@jax.jit
def _lod_interp(px, py, pz, feat_lin):
    mesh = plsc.VectorSubcoreMesh(core_axis_name="c", subcore_axis_name="s")
    run = pl.kernel(
        _tec_body,
        out_type=(
            jax.ShapeDtypeStruct((NPTS * FEAT // 128, 128), jnp.float32),
            jax.ShapeDtypeStruct((NC, RES ** 3, FEAT), jnp.float32),
        ),
        mesh=mesh,
        compiler_params=pltpu.CompilerParams(
            needs_layout_passes=False,
            use_tc_tiling_on_sc=False,
        ),
        scratch_types=[
            pltpu.VMEM((CHUNK,), jnp.float32),
            pltpu.VMEM((CHUNK,), jnp.float32),
            pltpu.VMEM((CHUNK,), jnp.float32),
            pltpu.VMEM((2, CHUNK), jnp.float32),
            pltpu.VMEM((2, CHUNK), jnp.float32),
            pltpu.VMEM((2, CHUNK), jnp.float32),
            pltpu.VMEM((2, 8, NSUB, SUB), jnp.int32),
            pltpu.VMEM((2, 8, CHUNK, FEAT), jnp.float32),
            pltpu.VMEM((CHUNK * FEAT // 128, 128), jnp.float32),
            pltpu.VMEM((STAGE_K, 128), jnp.float32),
            pltpu.VMEM((STAGE_K * 128 // FEAT, FEAT), jnp.float32),
            pltpu.SemaphoreType.DMA,
            pltpu.SemaphoreType.DMA,
        ],
    )
    return run(px, py, pz, feat_lin)


def kernel(pts, features):
    n = pts.shape[0]
    v = features.shape[0]
    feat_lin = (features.reshape(v // 128, 128, FEAT)
                .transpose(0, 2, 1)
                .reshape(v * FEAT // 128, 128))
    out128, _ = _lod_interp(pts[:, 0], pts[:, 1], pts[:, 2], feat_lin)
    return (out128.reshape(n // 128, FEAT, 128)
            .transpose(0, 2, 1)
            .reshape(n, FEAT))
